# baseline (speedup 1.0000x reference)
import functools

import jax
import jax.numpy as jnp
from jax import lax
from jax.experimental import pallas as pl
from jax.experimental.pallas import tpu as pltpu
from jax.experimental.pallas import tpu_sc as plsc

N = 10000
E = 320000
D = 128

NC = 2
NS = 16
NW = NC * NS
L = 16

NP = 10240
SLICE = NP // NS
ET = E // NW
CHUNK = 128
CT = 80
CTP = CT + 1
EP = NW * CTP * CHUNK

_mesh = plsc.VectorSubcoreMesh(core_axis_name="c", subcore_axis_name="s")
_sc_params = pltpu.CompilerParams(needs_layout_passes=False)


def _zero_1d(ref, n):
    zeros = jnp.zeros((L,), jnp.float32)

    def body(i, _):
        ref[pl.ds(i * L, L)] = zeros
        return 0

    lax.fori_loop(0, n // L, body, 0)


@functools.partial(
    pl.kernel,
    out_type=jax.ShapeDtypeStruct((2, NW, NP), jnp.float32),
    mesh=_mesh,
    compiler_params=_sc_params,
    scratch_types=[
        pltpu.VMEM((NP,), jnp.float32),
        pltpu.VMEM((NP,), jnp.float32),
        pltpu.VMEM((ET,), jnp.int32),
        pltpu.VMEM((ET,), jnp.int32),
    ],
)
def _sc_degrees(src_hbm, dst_hbm, out_hbm, ho, hi, sb, db):
    cid = lax.axis_index("c")
    sid = lax.axis_index("s")
    w = cid * NS + sid

    _zero_1d(ho, NP)
    _zero_1d(hi, NP)
    pltpu.sync_copy(src_hbm.at[pl.ds(w * ET, ET)], sb)
    pltpu.sync_copy(dst_hbm.at[pl.ds(w * ET, ET)], db)

    ones = jnp.ones((L,), jnp.float32)

    def hist_body(j, _):
        si = sb[pl.ds(j * L, L)]
        plsc.addupdate_scatter(ho, [si], ones)
        di = db[pl.ds(j * L, L)]
        plsc.addupdate_scatter(hi, [di], ones)
        return 0

    lax.fori_loop(0, ET // L, hist_body, 0)

    pltpu.sync_copy(ho, out_hbm.at[0, w])
    pltpu.sync_copy(hi, out_hbm.at[1, w])


@functools.partial(
    pl.kernel,
    out_type=jax.ShapeDtypeStruct((NW, NP), jnp.float32),
    mesh=_mesh,
    compiler_params=_sc_params,
    scratch_types=[
        pltpu.VMEM((ET,), jnp.int32),
        pltpu.VMEM((ET,), jnp.int32),
        pltpu.VMEM((ET,), jnp.float32),
        pltpu.VMEM((NP,), jnp.float32),
        pltpu.VMEM((NP,), jnp.float32),
    ],
)
def _sc_ssum(src_hbm, dst_hbm, ew_hbm, nin_hbm, s_out, sb, db, eb, ninl, sl):
    cid = lax.axis_index("c")
    sid = lax.axis_index("s")
    w = cid * NS + sid

    _zero_1d(sl, NP)
    pltpu.sync_copy(src_hbm.at[pl.ds(w * ET, ET)], sb)
    pltpu.sync_copy(dst_hbm.at[pl.ds(w * ET, ET)], db)
    pltpu.sync_copy(ew_hbm.at[pl.ds(w * ET, ET)], eb)
    pltpu.sync_copy(nin_hbm, ninl)

    def body(j, _):
        di = db[pl.ds(j * L, L)]
        ni16 = plsc.load_gather(ninl, [di])
        ew16 = eb[pl.ds(j * L, L)]
        si = sb[pl.ds(j * L, L)]
        plsc.addupdate_scatter(sl, [si], ew16 * ni16)
        return 0

    lax.fori_loop(0, ET // L, body, 0)
    pltpu.sync_copy(sl, s_out.at[w])


def _tc_prep_body(do_ref, di_ref, x_ref, w1_ref, no_ref, ni_ref, hw_ref):
    dego = jnp.sum(do_ref[...], axis=1, keepdims=True)
    degi = jnp.sum(di_ref[...], axis=1, keepdims=True)
    no = lax.rsqrt(jnp.clip(dego, 1.0, None))
    ni = lax.rsqrt(jnp.clip(degi, 1.0, None))
    no_ref[...] = no
    ni_ref[...] = ni
    h = x_ref[...] * no[:N]
    hw_ref[...] = jnp.dot(h, w1_ref[...], preferred_element_type=jnp.float32)


def _tc_prep(d_o, d_i, x, w1):
    return pl.pallas_call(
        _tc_prep_body,
        out_shape=(
            jax.ShapeDtypeStruct((NP, 1), jnp.float32),
            jax.ShapeDtypeStruct((NP, 1), jnp.float32),
            jax.ShapeDtypeStruct((N, D), jnp.float32),
        ),
    )(d_o, d_i, x, w1)


@functools.partial(
    pl.kernel,
    out_type=jax.ShapeDtypeStruct((NC, NP, D), jnp.float32),
    mesh=_mesh,
    compiler_params=_sc_params,
    scratch_types=[
        pltpu.VMEM((2, 4, CHUNK), jnp.int32),
        pltpu.VMEM((2, CHUNK, D), jnp.float32),
        pltpu.VMEM_SHARED((NP, D), jnp.float32),
        pltpu.SemaphoreType.DMA,
        pltpu.SemaphoreType.DMA,
        pltpu.SemaphoreType.DMA,
        pltpu.SemaphoreType.DMA,
    ],
)
def _sc_msgpass(hw_hbm, idxp, agg_out, ibuf, rows, agg_sh,
                gsem0, gsem1, ssem0, ssem1):
    cid = lax.axis_index("c")
    sid = lax.axis_index("s")
    w = cid * NS + sid
    lo = sid * SLICE

    zeros = jnp.zeros((L,), jnp.float32)

    def zbody(e, _):
        for v in range(D // L):
            rows[0, e, pl.ds(v * L, L)] = zeros
            rows[1, e, pl.ds(v * L, L)] = zeros
        return 0

    lax.fori_loop(0, CHUNK, zbody, 0)
    izeros = jnp.zeros((L,), jnp.int32)
    for k in range(CHUNK // L):
        ibuf[1, 1, pl.ds(k * L, L)] = izeros
    for k in range(SLICE // CHUNK):
        pltpu.sync_copy(rows.at[0],
                        agg_sh.at[pl.ds(sid * SLICE + k * CHUNK, CHUNK)])
    plsc.subcore_barrier()

    pltpu.sync_copy(idxp.at[w, 0], ibuf.at[0])
    pltpu.async_copy(hw_hbm.at[ibuf.at[0, 0]], rows.at[0], gsem0)

    def chunk_body(c, _):
        b = lax.rem(c, 2)
        nb = 1 - b
        pltpu.make_async_copy(
            hw_hbm.at[ibuf.at[b, 0]], rows.at[b], gsem0).wait()
        pltpu.sync_copy(idxp.at[w, c + 1], ibuf.at[nb])
        pltpu.async_copy(hw_hbm.at[ibuf.at[nb, 0]], rows.at[nb], gsem0)

        def mbody(j, _):
            ew16 = plsc.bitcast(ibuf[b, 2, pl.ds(j * L, L)], jnp.float32)
            for k in range(L):
                e = j * L + k
                wv = jnp.broadcast_to(ew16[k], (L,))
                for v in range(D // L):
                    rows[b, e, pl.ds(v * L, L)] = (
                        rows[b, e, pl.ds(v * L, L)] * wv)
            return 0

        lax.fori_loop(0, CHUNK // L, mbody, 0)

        pltpu.sync_copy(rows.at[b], agg_sh.at[ibuf.at[b, 1]], add=True)
        return 0

    lax.fori_loop(0, CT, chunk_body, 0)
    pltpu.make_async_copy(
        hw_hbm.at[ibuf.at[0, 0]], rows.at[0], gsem0).wait()
    plsc.subcore_barrier()

    pltpu.sync_copy(agg_sh.at[pl.ds(lo, SLICE)],
                    agg_out.at[cid, pl.ds(lo, SLICE)])


def _tc_finish_body(a0_ref, a1_ref, ni_ref, sp_ref, no_ref, b1_ref, w2_ref,
                    b2_ref, out_ref):
    a = a0_ref[...] + a1_ref[...]
    h1 = jnp.maximum(a[:N] * ni_ref[...][:N] + b1_ref[...], 0.0)
    s = jnp.sum(sp_ref[...], axis=1, keepdims=True)
    wgt = (s * no_ref[...])[:N]
    u = jnp.sum(wgt * h1, axis=0, keepdims=True)
    out_ref[...] = b2_ref[...] + jnp.dot(
        u, w2_ref[...], preferred_element_type=jnp.float32) * (1.0 / N)


def _tc_finish(a0, a1, ni, sp, no, b1, w2, b2):
    return pl.pallas_call(
        _tc_finish_body,
        out_shape=jax.ShapeDtypeStruct((1, D), jnp.float32),
    )(a0, a1, ni, sp, no, b1, w2, b2)


def kernel(x, edge_index, edge_weight, W1, b1, W2, b2):
    src = edge_index[0]
    dst = edge_index[1]

    degs = _sc_degrees(src, dst)
    no, ni, hw = _tc_prep(degs[0].T, degs[1].T, x, W1)

    pad = NW * CT * CHUNK - E
    ew_bits = lax.bitcast_convert_type(
        jnp.pad(edge_weight, (0, pad)), jnp.int32)
    idxp = jnp.stack([
        jnp.pad(src, (0, pad)),
        jnp.pad(dst, (0, pad)),
        ew_bits,
        jnp.zeros((NW * CT * CHUNK,), jnp.int32),
    ])
    idxp = idxp.reshape(4, NW, CT, CHUNK).transpose(1, 2, 0, 3)
    idxp = jnp.concatenate(
        [idxp, jnp.zeros((NW, 1, 4, CHUNK), jnp.int32)], axis=1)

    aggp = _sc_msgpass(hw, idxp)
    sp = _sc_ssum(src, dst, edge_weight, ni.reshape(NP))
    out = _tc_finish(aggp[0], aggp[1], ni, sp.T, no,
                     b1.reshape(1, D), W2, b2.reshape(1, D))
    return out


# speedup vs baseline: 1.6699x; 1.6699x over previous
import functools

import jax
import jax.numpy as jnp
from jax import lax
from jax.experimental import pallas as pl
from jax.experimental.pallas import tpu as pltpu
from jax.experimental.pallas import tpu_sc as plsc

N = 10000
E = 320000
D = 128

NC = 2
NS = 16
NW = NC * NS
L = 16

NP = 10240
SLICE = NP // NS
ET = E // NW
CHUNK = 128
CT = -(-E // (NW * CHUNK))
EP = NW * CT * CHUNK

_mesh = plsc.VectorSubcoreMesh(core_axis_name="c", subcore_axis_name="s")
_sc_params = pltpu.CompilerParams(needs_layout_passes=False)


def _zero_1d(ref, n):
    zeros = jnp.zeros((L,), jnp.float32)

    def body(i, _):
        ref[pl.ds(i * L, L)] = zeros
        return 0

    lax.fori_loop(0, n // L, body, 0)


@functools.partial(
    pl.kernel,
    out_type=jax.ShapeDtypeStruct((2, NW, NP), jnp.float32),
    mesh=_mesh,
    compiler_params=_sc_params,
    scratch_types=[
        pltpu.VMEM((NP,), jnp.float32),
        pltpu.VMEM((NP,), jnp.float32),
        pltpu.VMEM((ET,), jnp.int32),
        pltpu.VMEM((ET,), jnp.int32),
    ],
)
def _sc_degrees(src_hbm, dst_hbm, out_hbm, ho, hi, sb, db):
    cid = lax.axis_index("c")
    sid = lax.axis_index("s")
    w = cid * NS + sid

    _zero_1d(ho, NP)
    _zero_1d(hi, NP)
    pltpu.sync_copy(src_hbm.at[pl.ds(w * ET, ET)], sb)
    pltpu.sync_copy(dst_hbm.at[pl.ds(w * ET, ET)], db)

    ones = jnp.ones((L,), jnp.float32)

    def hist_body(j, _):
        si = sb[pl.ds(j * L, L)]
        plsc.addupdate_scatter(ho, [si], ones)
        di = db[pl.ds(j * L, L)]
        plsc.addupdate_scatter(hi, [di], ones)
        return 0

    lax.fori_loop(0, ET // L, hist_body, 0)

    pltpu.sync_copy(ho, out_hbm.at[0, w])
    pltpu.sync_copy(hi, out_hbm.at[1, w])


@functools.partial(
    pl.kernel,
    out_type=jax.ShapeDtypeStruct((NW, NP), jnp.float32),
    mesh=_mesh,
    compiler_params=_sc_params,
    scratch_types=[
        pltpu.VMEM((ET,), jnp.int32),
        pltpu.VMEM((ET,), jnp.int32),
        pltpu.VMEM((ET,), jnp.float32),
        pltpu.VMEM((NP,), jnp.float32),
        pltpu.VMEM((NP,), jnp.float32),
    ],
)
def _sc_ssum(src_hbm, dst_hbm, ew_hbm, nin_hbm, s_out, sb, db, eb, ninl, sl):
    cid = lax.axis_index("c")
    sid = lax.axis_index("s")
    w = cid * NS + sid

    _zero_1d(sl, NP)
    pltpu.sync_copy(src_hbm.at[pl.ds(w * ET, ET)], sb)
    pltpu.sync_copy(dst_hbm.at[pl.ds(w * ET, ET)], db)
    pltpu.sync_copy(ew_hbm.at[pl.ds(w * ET, ET)], eb)
    pltpu.sync_copy(nin_hbm, ninl)

    def body(j, _):
        di = db[pl.ds(j * L, L)]
        ni16 = plsc.load_gather(ninl, [di])
        ew16 = eb[pl.ds(j * L, L)]
        si = sb[pl.ds(j * L, L)]
        plsc.addupdate_scatter(sl, [si], ew16 * ni16)
        return 0

    lax.fori_loop(0, ET // L, body, 0)
    pltpu.sync_copy(sl, s_out.at[w])


def _tc_prep_body(do_ref, di_ref, x_ref, w1_ref, nor_ref, ni_ref, hw_ref):
    dego = jnp.sum(do_ref[...], axis=0, keepdims=True)
    degi = jnp.sum(di_ref[...], axis=0, keepdims=True)
    no_row = lax.rsqrt(jnp.clip(dego, 1.0, None))
    ni_row = lax.rsqrt(jnp.clip(degi, 1.0, None))
    nor_ref[...] = no_row
    ni_ref[...] = ni_row.T
    h = x_ref[...] * no_row.T[:N]
    hw_ref[...] = jnp.dot(h, w1_ref[...], preferred_element_type=jnp.float32)


def _tc_prep(d_o, d_i, x, w1):
    return pl.pallas_call(
        _tc_prep_body,
        out_shape=(
            jax.ShapeDtypeStruct((1, NP), jnp.float32),
            jax.ShapeDtypeStruct((NP, 1), jnp.float32),
            jax.ShapeDtypeStruct((N, D), jnp.float32),
        ),
    )(d_o, d_i, x, w1)


@functools.partial(
    pl.kernel,
    out_type=jax.ShapeDtypeStruct((NC, NP, D), jnp.float32),
    mesh=_mesh,
    compiler_params=_sc_params,
    scratch_types=[
        pltpu.VMEM((2, 4, CHUNK), jnp.int32),
        pltpu.VMEM((2, CHUNK, D), jnp.float32),
        pltpu.VMEM_SHARED((NP, D), jnp.float32),
        pltpu.SemaphoreType.DMA,
        pltpu.SemaphoreType.DMA,
        pltpu.SemaphoreType.DMA,
        pltpu.SemaphoreType.DMA,
    ],
)
def _sc_msgpass(hw_hbm, idxp, agg_out, ibuf, rows, agg_sh,
                gsem0, gsem1, ssem0, ssem1):
    cid = lax.axis_index("c")
    sid = lax.axis_index("s")
    w = cid * NS + sid
    lo = sid * SLICE

    zeros = jnp.zeros((L,), jnp.float32)

    def zbody(e, _):
        for v in range(D // L):
            rows[0, e, pl.ds(v * L, L)] = zeros
            rows[1, e, pl.ds(v * L, L)] = zeros
        return 0

    lax.fori_loop(0, CHUNK, zbody, 0)
    izeros = jnp.zeros((L,), jnp.int32)
    for k in range(CHUNK // L):
        ibuf[1, 1, pl.ds(k * L, L)] = izeros
    for k in range(SLICE // CHUNK):
        pltpu.sync_copy(rows.at[0],
                        agg_sh.at[pl.ds(sid * SLICE + k * CHUNK, CHUNK)])
    plsc.subcore_barrier()

    def chunk_body(c, _):
        pltpu.sync_copy(idxp.at[w, c], ibuf.at[0])
        pltpu.async_copy(hw_hbm.at[ibuf.at[0, 0]], rows.at[0], gsem0).wait()

        def mbody(j, _):
            ew16 = plsc.bitcast(ibuf[0, 2, pl.ds(j * L, L)], jnp.float32)
            for k in range(L):
                e = j * L + k
                wv = jnp.broadcast_to(ew16[k], (L,))
                for v in range(D // L):
                    rows[0, e, pl.ds(v * L, L)] = (
                        rows[0, e, pl.ds(v * L, L)] * wv)
            return 0

        lax.fori_loop(0, CHUNK // L, mbody, 0)

        pltpu.sync_copy(rows.at[0], agg_sh.at[ibuf.at[0, 1]], add=True)
        return 0

    lax.fori_loop(0, CT, chunk_body, 0)
    plsc.subcore_barrier()

    pltpu.sync_copy(agg_sh.at[pl.ds(lo, SLICE)],
                    agg_out.at[cid, pl.ds(lo, SLICE)])


def _tc_finish_body(a0_ref, a1_ref, ni_ref, sp_ref, nor_ref, b1_ref, w2_ref,
                    b2_ref, out_ref):
    a = a0_ref[...] + a1_ref[...]
    h1 = jnp.maximum(a[:N] * ni_ref[...][:N] + b1_ref[...], 0.0)
    s_row = jnp.sum(sp_ref[...], axis=0, keepdims=True)
    w_row = (s_row * nor_ref[...])[:, :N]
    u = jnp.dot(w_row, h1, preferred_element_type=jnp.float32)
    out_ref[...] = b2_ref[...] + jnp.dot(
        u, w2_ref[...], preferred_element_type=jnp.float32) * (1.0 / N)


def _tc_finish(a0, a1, ni, sp, no_row, b1, w2, b2):
    return pl.pallas_call(
        _tc_finish_body,
        out_shape=jax.ShapeDtypeStruct((1, D), jnp.float32),
    )(a0, a1, ni, sp, no_row, b1, w2, b2)


def kernel(x, edge_index, edge_weight, W1, b1, W2, b2):
    src = edge_index[0]
    dst = edge_index[1]

    degs = _sc_degrees(src, dst)
    no_row, ni, hw = _tc_prep(degs[0], degs[1], x, W1)

    pad = EP - E
    ew_bits = lax.bitcast_convert_type(
        jnp.pad(edge_weight, (0, pad)), jnp.int32)
    idxp = jnp.stack([
        jnp.pad(src, (0, pad)).reshape(NW, CT, CHUNK),
        jnp.pad(dst, (0, pad)).reshape(NW, CT, CHUNK),
        ew_bits.reshape(NW, CT, CHUNK),
        jnp.zeros((NW, CT, CHUNK), jnp.int32),
    ], axis=2)

    aggp = _sc_msgpass(hw, idxp)
    sp = _sc_ssum(src, dst, edge_weight, ni.reshape(NP))
    out = _tc_finish(aggp[0], aggp[1], ni, sp, no_row,
                     b1.reshape(1, D), W2, b2.reshape(1, D))
    return out


# speedup vs baseline: 1.6714x; 1.0009x over previous
import functools

import jax
import jax.numpy as jnp
from jax import lax
from jax.experimental import pallas as pl
from jax.experimental.pallas import tpu as pltpu
from jax.experimental.pallas import tpu_sc as plsc

N = 10000
E = 320000
D = 128

NC = 2
NS = 16
NW = NC * NS
L = 16

NP = 10240
SLICE = NP // NS
ET = E // NW
CHUNK = 128
CT = -(-E // (NW * CHUNK))
EP = NW * CT * CHUNK

_mesh = plsc.VectorSubcoreMesh(core_axis_name="c", subcore_axis_name="s")
_sc_params = pltpu.CompilerParams(needs_layout_passes=False)


def _zero_1d(ref, n):
    zeros = jnp.zeros((L,), jnp.float32)

    def body(i, _):
        ref[pl.ds(i * L, L)] = zeros
        return 0

    lax.fori_loop(0, n // L, body, 0)


@functools.partial(
    pl.kernel,
    out_type=jax.ShapeDtypeStruct((2, NW, NP), jnp.float32),
    mesh=_mesh,
    compiler_params=_sc_params,
    scratch_types=[
        pltpu.VMEM((NP,), jnp.float32),
        pltpu.VMEM((NP,), jnp.float32),
        pltpu.VMEM((ET,), jnp.int32),
        pltpu.VMEM((ET,), jnp.int32),
    ],
)
def _sc_degrees(src_hbm, dst_hbm, out_hbm, ho, hi, sb, db):
    cid = lax.axis_index("c")
    sid = lax.axis_index("s")
    w = cid * NS + sid

    _zero_1d(ho, NP)
    _zero_1d(hi, NP)
    pltpu.sync_copy(src_hbm.at[pl.ds(w * ET, ET)], sb)
    pltpu.sync_copy(dst_hbm.at[pl.ds(w * ET, ET)], db)

    ones = jnp.ones((L,), jnp.float32)

    def hist_body(j, _):
        si = sb[pl.ds(j * L, L)]
        plsc.addupdate_scatter(ho, [si], ones)
        di = db[pl.ds(j * L, L)]
        plsc.addupdate_scatter(hi, [di], ones)
        return 0

    lax.fori_loop(0, ET // L, hist_body, 0)

    pltpu.sync_copy(ho, out_hbm.at[0, w])
    pltpu.sync_copy(hi, out_hbm.at[1, w])


@functools.partial(
    pl.kernel,
    out_type=jax.ShapeDtypeStruct((NW, NP), jnp.float32),
    mesh=_mesh,
    compiler_params=_sc_params,
    scratch_types=[
        pltpu.VMEM((ET,), jnp.int32),
        pltpu.VMEM((ET,), jnp.int32),
        pltpu.VMEM((ET,), jnp.float32),
        pltpu.VMEM((NP,), jnp.float32),
        pltpu.VMEM((NP,), jnp.float32),
    ],
)
def _sc_ssum(src_hbm, dst_hbm, ew_hbm, nin_hbm, s_out, sb, db, eb, ninl, sl):
    cid = lax.axis_index("c")
    sid = lax.axis_index("s")
    w = cid * NS + sid

    _zero_1d(sl, NP)
    pltpu.sync_copy(src_hbm.at[pl.ds(w * ET, ET)], sb)
    pltpu.sync_copy(dst_hbm.at[pl.ds(w * ET, ET)], db)
    pltpu.sync_copy(ew_hbm.at[pl.ds(w * ET, ET)], eb)
    pltpu.sync_copy(nin_hbm, ninl)

    def body(j, _):
        di = db[pl.ds(j * L, L)]
        ni16 = plsc.load_gather(ninl, [di])
        ew16 = eb[pl.ds(j * L, L)]
        si = sb[pl.ds(j * L, L)]
        plsc.addupdate_scatter(sl, [si], ew16 * ni16)
        return 0

    lax.fori_loop(0, ET // L, body, 0)
    pltpu.sync_copy(sl, s_out.at[w])


def _tc_prep_body(do_ref, di_ref, x_ref, w1_ref, nor_ref, ni_ref, hw_ref):
    dego = jnp.sum(do_ref[...], axis=0, keepdims=True)
    degi = jnp.sum(di_ref[...], axis=0, keepdims=True)
    no_row = lax.rsqrt(jnp.clip(dego, 1.0, None))
    ni_row = lax.rsqrt(jnp.clip(degi, 1.0, None))
    nor_ref[...] = no_row
    ni_ref[...] = ni_row.T
    h = x_ref[...] * no_row.T[:N]
    hw_ref[...] = jnp.dot(h, w1_ref[...], preferred_element_type=jnp.float32)


def _tc_prep(d_o, d_i, x, w1):
    return pl.pallas_call(
        _tc_prep_body,
        out_shape=(
            jax.ShapeDtypeStruct((1, NP), jnp.float32),
            jax.ShapeDtypeStruct((NP, 1), jnp.float32),
            jax.ShapeDtypeStruct((N, D), jnp.float32),
        ),
    )(d_o, d_i, x, w1)


@functools.partial(
    pl.kernel,
    out_type=jax.ShapeDtypeStruct((NC, NP, D), jnp.float32),
    mesh=_mesh,
    compiler_params=_sc_params,
    scratch_types=[
        pltpu.VMEM((2, 4, CHUNK), jnp.int32),
        pltpu.VMEM((2, CHUNK, D), jnp.float32),
        pltpu.VMEM_SHARED((NP, D), jnp.float32),
        pltpu.SemaphoreType.DMA,
        pltpu.SemaphoreType.DMA,
        pltpu.SemaphoreType.DMA,
        pltpu.SemaphoreType.DMA,
    ],
)
def _sc_msgpass(hw_hbm, idxp, agg_out, ibuf, rows, agg_sh,
                gsem0, gsem1, ssem0, ssem1):
    cid = lax.axis_index("c")
    sid = lax.axis_index("s")
    w = cid * NS + sid
    lo = sid * SLICE

    zeros = jnp.zeros((L,), jnp.float32)

    def zbody(e, _):
        for v in range(D // L):
            rows[0, e, pl.ds(v * L, L)] = zeros
            rows[1, e, pl.ds(v * L, L)] = zeros
        return 0

    lax.fori_loop(0, CHUNK, zbody, 0)
    izeros = jnp.zeros((L,), jnp.int32)
    for k in range(CHUNK // L):
        ibuf[1, 1, pl.ds(k * L, L)] = izeros
    for k in range(SLICE // CHUNK):
        pltpu.sync_copy(rows.at[0],
                        agg_sh.at[pl.ds(sid * SLICE + k * CHUNK, CHUNK)])
    plsc.subcore_barrier()

    def chunk_body(c, _):
        pltpu.sync_copy(idxp.at[w, c], ibuf.at[0])
        pltpu.async_copy(hw_hbm.at[ibuf.at[0, 0]], rows.at[0], gsem0).wait()

        @plsc.parallel_loop(0, CHUNK // L, unroll=2)
        def mbody(j):
            ew16 = plsc.bitcast(ibuf[0, 2, pl.ds(j * L, L)], jnp.float32)
            for k in range(L):
                e = j * L + k
                wv = jnp.broadcast_to(ew16[k], (L,))
                for v in range(D // L):
                    rows[0, e, pl.ds(v * L, L)] = (
                        rows[0, e, pl.ds(v * L, L)] * wv)

        pltpu.sync_copy(rows.at[0], agg_sh.at[ibuf.at[0, 1]], add=True)
        return 0

    lax.fori_loop(0, CT, chunk_body, 0)
    plsc.subcore_barrier()

    pltpu.sync_copy(agg_sh.at[pl.ds(lo, SLICE)],
                    agg_out.at[cid, pl.ds(lo, SLICE)])


def _tc_finish_body(a0_ref, a1_ref, ni_ref, sp_ref, nor_ref, b1_ref, w2_ref,
                    b2_ref, out_ref):
    a = a0_ref[...] + a1_ref[...]
    h1 = jnp.maximum(a[:N] * ni_ref[...][:N] + b1_ref[...], 0.0)
    s_row = jnp.sum(sp_ref[...], axis=0, keepdims=True)
    w_row = (s_row * nor_ref[...])[:, :N]
    u = jnp.dot(w_row, h1, preferred_element_type=jnp.float32)
    out_ref[...] = b2_ref[...] + jnp.dot(
        u, w2_ref[...], preferred_element_type=jnp.float32) * (1.0 / N)


def _tc_finish(a0, a1, ni, sp, no_row, b1, w2, b2):
    return pl.pallas_call(
        _tc_finish_body,
        out_shape=jax.ShapeDtypeStruct((1, D), jnp.float32),
    )(a0, a1, ni, sp, no_row, b1, w2, b2)


def kernel(x, edge_index, edge_weight, W1, b1, W2, b2):
    src = edge_index[0]
    dst = edge_index[1]

    degs = _sc_degrees(src, dst)
    no_row, ni, hw = _tc_prep(degs[0], degs[1], x, W1)

    pad = EP - E
    ew_bits = lax.bitcast_convert_type(
        jnp.pad(edge_weight, (0, pad)), jnp.int32)
    idxp = jnp.stack([
        jnp.pad(src, (0, pad)).reshape(NW, CT, CHUNK),
        jnp.pad(dst, (0, pad)).reshape(NW, CT, CHUNK),
        ew_bits.reshape(NW, CT, CHUNK),
        jnp.zeros((NW, CT, CHUNK), jnp.int32),
    ], axis=2)

    aggp = _sc_msgpass(hw, idxp)
    sp = _sc_ssum(src, dst, edge_weight, ni.reshape(NP))
    out = _tc_finish(aggp[0], aggp[1], ni, sp, no_row,
                     b1.reshape(1, D), W2, b2.reshape(1, D))
    return out


# speedup vs baseline: 1.6736x; 1.0013x over previous
import functools

import jax
import jax.numpy as jnp
from jax import lax
from jax.experimental import pallas as pl
from jax.experimental.pallas import tpu as pltpu
from jax.experimental.pallas import tpu_sc as plsc

N = 10000
E = 320000
D = 128

NC = 2
NS = 16
NW = NC * NS
L = 16

NP = 10240
SLICE = NP // NS
ET = E // NW
CHUNK = 128
CT = -(-E // (NW * CHUNK))
EP = NW * CT * CHUNK

_mesh = plsc.VectorSubcoreMesh(core_axis_name="c", subcore_axis_name="s")
_sc_params = pltpu.CompilerParams(needs_layout_passes=False)


def _zero_1d(ref, n):
    zeros = jnp.zeros((L,), jnp.float32)

    def body(i, _):
        ref[pl.ds(i * L, L)] = zeros
        return 0

    lax.fori_loop(0, n // L, body, 0)


@functools.partial(
    pl.kernel,
    out_type=jax.ShapeDtypeStruct((2, NW, NP), jnp.float32),
    mesh=_mesh,
    compiler_params=_sc_params,
    scratch_types=[
        pltpu.VMEM((NP,), jnp.float32),
        pltpu.VMEM((NP,), jnp.float32),
        pltpu.VMEM((ET,), jnp.int32),
        pltpu.VMEM((ET,), jnp.int32),
    ],
)
def _sc_degrees(src_hbm, dst_hbm, out_hbm, ho, hi, sb, db):
    cid = lax.axis_index("c")
    sid = lax.axis_index("s")
    w = cid * NS + sid

    _zero_1d(ho, NP)
    _zero_1d(hi, NP)
    pltpu.sync_copy(src_hbm.at[pl.ds(w * ET, ET)], sb)
    pltpu.sync_copy(dst_hbm.at[pl.ds(w * ET, ET)], db)

    ones = jnp.ones((L,), jnp.float32)

    def hist_body(j, _):
        si = sb[pl.ds(j * L, L)]
        plsc.addupdate_scatter(ho, [si], ones)
        di = db[pl.ds(j * L, L)]
        plsc.addupdate_scatter(hi, [di], ones)
        return 0

    lax.fori_loop(0, ET // L, hist_body, 0)

    pltpu.sync_copy(ho, out_hbm.at[0, w])
    pltpu.sync_copy(hi, out_hbm.at[1, w])


@functools.partial(
    pl.kernel,
    out_type=jax.ShapeDtypeStruct((NW, NP), jnp.float32),
    mesh=_mesh,
    compiler_params=_sc_params,
    scratch_types=[
        pltpu.VMEM((ET,), jnp.int32),
        pltpu.VMEM((ET,), jnp.int32),
        pltpu.VMEM((ET,), jnp.float32),
        pltpu.VMEM((NP,), jnp.float32),
        pltpu.VMEM((NP,), jnp.float32),
    ],
)
def _sc_ssum(src_hbm, dst_hbm, ew_hbm, nin_hbm, s_out, sb, db, eb, ninl, sl):
    cid = lax.axis_index("c")
    sid = lax.axis_index("s")
    w = cid * NS + sid

    _zero_1d(sl, NP)
    pltpu.sync_copy(src_hbm.at[pl.ds(w * ET, ET)], sb)
    pltpu.sync_copy(dst_hbm.at[pl.ds(w * ET, ET)], db)
    pltpu.sync_copy(ew_hbm.at[pl.ds(w * ET, ET)], eb)
    pltpu.sync_copy(nin_hbm, ninl)

    def body(j, _):
        di = db[pl.ds(j * L, L)]
        ni16 = plsc.load_gather(ninl, [di])
        ew16 = eb[pl.ds(j * L, L)]
        si = sb[pl.ds(j * L, L)]
        plsc.addupdate_scatter(sl, [si], ew16 * ni16)
        return 0

    lax.fori_loop(0, ET // L, body, 0)
    pltpu.sync_copy(sl, s_out.at[w])


def _tc_prep_body(do_ref, di_ref, x_ref, w1_ref, nor_ref, ni_ref, hw_ref):
    dego = jnp.sum(do_ref[...], axis=0, keepdims=True)
    degi = jnp.sum(di_ref[...], axis=0, keepdims=True)
    no_row = lax.rsqrt(jnp.clip(dego, 1.0, None))
    ni_row = lax.rsqrt(jnp.clip(degi, 1.0, None))
    nor_ref[...] = no_row
    ni_ref[...] = ni_row.T
    h = x_ref[...] * no_row.T[:N]
    hw_ref[...] = jnp.dot(h, w1_ref[...], preferred_element_type=jnp.float32)


def _tc_prep(d_o, d_i, x, w1):
    return pl.pallas_call(
        _tc_prep_body,
        out_shape=(
            jax.ShapeDtypeStruct((1, NP), jnp.float32),
            jax.ShapeDtypeStruct((NP, 1), jnp.float32),
            jax.ShapeDtypeStruct((N, D), jnp.float32),
        ),
    )(d_o, d_i, x, w1)


@functools.partial(
    pl.kernel,
    out_type=jax.ShapeDtypeStruct((NC, NP, D), jnp.float32),
    mesh=_mesh,
    compiler_params=_sc_params,
    scratch_types=[
        pltpu.VMEM((1, 4, CHUNK), jnp.int32),
        pltpu.VMEM((1, CHUNK, D), jnp.float32),
        pltpu.VMEM_SHARED((NP, D), jnp.float32),
        pltpu.SemaphoreType.DMA,
    ],
)
def _sc_msgpass(hw_hbm, idxp, agg_out, ibuf, rows, agg_sh, gsem0):
    cid = lax.axis_index("c")
    sid = lax.axis_index("s")
    w = cid * NS + sid
    lo = sid * SLICE

    zeros = jnp.zeros((L,), jnp.float32)

    def zbody(e, _):
        for v in range(D // L):
            rows[0, e, pl.ds(v * L, L)] = zeros
        return 0

    lax.fori_loop(0, CHUNK, zbody, 0)
    for k in range(SLICE // CHUNK):
        pltpu.sync_copy(rows.at[0],
                        agg_sh.at[pl.ds(sid * SLICE + k * CHUNK, CHUNK)])
    plsc.subcore_barrier()

    def chunk_body(c, _):
        pltpu.sync_copy(idxp.at[w, c], ibuf.at[0])
        pltpu.async_copy(hw_hbm.at[ibuf.at[0, 0]], rows.at[0], gsem0).wait()

        @plsc.parallel_loop(0, CHUNK // L, unroll=2)
        def mbody(j):
            ew16 = plsc.bitcast(ibuf[0, 2, pl.ds(j * L, L)], jnp.float32)
            for k in range(L):
                e = j * L + k
                wv = jnp.broadcast_to(ew16[k], (L,))
                for v in range(D // L):
                    rows[0, e, pl.ds(v * L, L)] = (
                        rows[0, e, pl.ds(v * L, L)] * wv)

        pltpu.sync_copy(rows.at[0], agg_sh.at[ibuf.at[0, 1]], add=True)
        return 0

    lax.fori_loop(0, CT, chunk_body, 0)
    plsc.subcore_barrier()

    pltpu.sync_copy(agg_sh.at[pl.ds(lo, SLICE)],
                    agg_out.at[cid, pl.ds(lo, SLICE)])


def _tc_finish_body(a0_ref, a1_ref, ni_ref, sp_ref, nor_ref, b1_ref, w2_ref,
                    b2_ref, out_ref):
    a = a0_ref[...] + a1_ref[...]
    h1 = jnp.maximum(a[:N] * ni_ref[...][:N] + b1_ref[...], 0.0)
    s_row = jnp.sum(sp_ref[...], axis=0, keepdims=True)
    w_row = (s_row * nor_ref[...])[:, :N]
    u = jnp.dot(w_row, h1, preferred_element_type=jnp.float32)
    out_ref[...] = b2_ref[...] + jnp.dot(
        u, w2_ref[...], preferred_element_type=jnp.float32) * (1.0 / N)


def _tc_finish(a0, a1, ni, sp, no_row, b1, w2, b2):
    return pl.pallas_call(
        _tc_finish_body,
        out_shape=jax.ShapeDtypeStruct((1, D), jnp.float32),
    )(a0, a1, ni, sp, no_row, b1, w2, b2)


def kernel(x, edge_index, edge_weight, W1, b1, W2, b2):
    src = edge_index[0]
    dst = edge_index[1]

    degs = _sc_degrees(src, dst)
    no_row, ni, hw = _tc_prep(degs[0], degs[1], x, W1)

    pad = EP - E
    ew_bits = lax.bitcast_convert_type(
        jnp.pad(edge_weight, (0, pad)), jnp.int32)
    idxp = jnp.stack([
        jnp.pad(src, (0, pad)).reshape(NW, CT, CHUNK),
        jnp.pad(dst, (0, pad)).reshape(NW, CT, CHUNK),
        ew_bits.reshape(NW, CT, CHUNK),
        jnp.zeros((NW, CT, CHUNK), jnp.int32),
    ], axis=2)

    aggp = _sc_msgpass(hw, idxp)
    sp = _sc_ssum(src, dst, edge_weight, ni.reshape(NP))
    out = _tc_finish(aggp[0], aggp[1], ni, sp, no_row,
                     b1.reshape(1, D), W2, b2.reshape(1, D))
    return out
